# baseline (speedup 1.0000x reference)
import functools

import jax
import jax.numpy as jnp
from jax import lax
from jax.experimental import pallas as pl
from jax.experimental.pallas import tpu as pltpu
from jax.experimental.pallas import tpu_sc as plsc

N = 10000
E = 320000
F = 128
N_PAD = 10240
NB = N_PAD // F
NW = 32
EPT = 10240
CHUNKS = EPT // F
NBUF = 2
DNBUF = 4
E_PAD = NW * EPT
DUMMY = N

_mesh = plsc.VectorSubcoreMesh(core_axis_name="c", subcore_axis_name="s")


@functools.partial(
    pl.kernel,
    out_type=jax.ShapeDtypeStruct((2, NB, F), jnp.float32),
    mesh=_mesh,
    scratch_types=[
        pltpu.VMEM((CHUNKS, F), jnp.int32),
        pltpu.VMEM((DNBUF, F), jnp.int32),
        pltpu.VMEM((DNBUF, F), jnp.int32),
        [pltpu.VMEM((F, F), jnp.float32)] * DNBUF,
        pltpu.VMEM((NB, F), jnp.float32),
        pltpu.VMEM_SHARED((F, F), jnp.float32),
        pltpu.VMEM_SHARED((NB, F), jnp.float32),
        [pltpu.SemaphoreType.DMA] * DNBUF,
    ],
)
def _deg_kernel(eye_hbm, dst_hbm, out_hbm,
                dst_v, cidx_v, ridx_v, rows, zb_v, eye_sh, deg_sh, sems):
    c = lax.axis_index("c")
    s = lax.axis_index("s")
    wid = c * 16 + s
    z16 = jnp.zeros((16,), jnp.float32)

    def zbody(i, carry):
        for k in range(8):
            zb_v[i, pl.ds(k * 16, 16)] = z16
        return carry

    lax.fori_loop(0, NB, zbody, 0)

    @pl.when(s == 0)
    def _():
        pltpu.sync_copy(zb_v, deg_sh)

    @pl.when(s == 1)
    def _():
        pltpu.sync_copy(eye_hbm, eye_sh)

    pltpu.sync_copy(dst_hbm.at[wid], dst_v)

    def setidx(chunk, b):
        for k in range(8):
            sl = pl.ds(k * 16, 16)
            d = dst_v[chunk, sl]
            cidx_v[b, sl] = lax.bitwise_and(d, F - 1)
            ridx_v[b, sl] = lax.shift_right_logical(d, 7)

    for b in range(DNBUF):
        setidx(b, b)
        pltpu.async_copy(eye_sh.at[cidx_v.at[b]], rows[b], sems[b])
    plsc.subcore_barrier()

    def body(g, carry):
        for b in range(DNBUF):
            chunk = g * DNBUF + b
            pltpu.make_async_copy(eye_sh.at[cidx_v.at[b]], rows[b],
                                  sems[b]).wait()
            pltpu.sync_copy(rows[b], deg_sh.at[ridx_v.at[b]], add=True)
            nxt = chunk + DNBUF

            @pl.when(nxt < CHUNKS)
            def _():
                setidx(nxt, b)
                pltpu.async_copy(eye_sh.at[cidx_v.at[b]], rows[b], sems[b])
        return carry

    lax.fori_loop(0, CHUNKS // DNBUF, body, 0)
    plsc.subcore_barrier()

    @pl.when(s == 0)
    def _():
        pltpu.sync_copy(deg_sh, out_hbm.at[c])


@functools.partial(
    pl.kernel,
    out_type=jax.ShapeDtypeStruct((2, N_PAD, F), jnp.float32),
    mesh=_mesh,
    scratch_types=[
        pltpu.VMEM((CHUNKS, F), jnp.int32),
        pltpu.VMEM((CHUNKS, F), jnp.int32),
        [pltpu.VMEM((F, F), jnp.float32)] * NBUF,
        pltpu.VMEM_SHARED((N_PAD, F), jnp.float32),
        [pltpu.SemaphoreType.DMA] * NBUF,
    ],
)
def _agg_kernel(table_hbm, src_hbm, dst_hbm, out_hbm,
                src_v, dst_v, rows, acc_sh, sems):
    c = lax.axis_index("c")
    s = lax.axis_index("s")
    wid = c * 16 + s
    z16 = jnp.zeros((16,), jnp.float32)

    def zbody(i, carry):
        for k in range(8):
            rows[0][i, pl.ds(k * 16, 16)] = z16
        return carry

    lax.fori_loop(0, F, zbody, 0)
    rows_per_tile = N_PAD // 16

    def zsh(k, carry):
        pltpu.sync_copy(rows[0],
                        acc_sh.at[pl.ds(s * rows_per_tile + k * F, F)])
        return carry

    lax.fori_loop(0, rows_per_tile // F, zsh, 0)
    pltpu.sync_copy(src_hbm.at[wid], src_v)
    pltpu.sync_copy(dst_hbm.at[wid], dst_v)
    plsc.subcore_barrier()

    def body(chunk, carry):
        pltpu.async_copy(table_hbm.at[src_v.at[chunk]], rows[0],
                         sems[0]).wait()
        pltpu.sync_copy(rows[0], acc_sh.at[dst_v.at[chunk]], add=True)
        return carry

    lax.fori_loop(0, CHUNKS, body, 0)
    plsc.subcore_barrier()
    pltpu.sync_copy(acc_sh.at[pl.ds(s * rows_per_tile, rows_per_tile)],
                    out_hbm.at[c, pl.ds(s * rows_per_tile, rows_per_tile)])


def _diag(dinv_row):
    row = lax.broadcasted_iota(jnp.int32, (F, F), 0)
    col = lax.broadcasted_iota(jnp.int32, (F, F), 1)
    return jnp.where(row == col, jnp.broadcast_to(dinv_row, (F, F)), 0.0)


def _mm(a, b):
    return jax.lax.dot_general(a, b, (((1,), (0,)), ((), ())),
                               preferred_element_type=jnp.float32)


def _tc1_body(x, dega, degb, w1, b1, wg0, bg0, dinv_o, h0_o, gs0_o):
    dinv = lax.rsqrt(dega[...] + degb[...] + 1.0)
    dinv_o[...] = dinv
    h0 = _mm(x[...], w1[...]) + b1[...]
    h0_o[...] = h0
    g0 = _mm(h0, wg0[...]) + bg0[...]
    gs0_o[...] = _mm(_diag(dinv[0]), g0)


def _tc2_body(s0a, s0b, gs0, h0, dinv, wg1, bg1, h01_o, gs1_o):
    d = _diag(dinv[...][0])
    h1 = jax.nn.relu(_mm(d, s0a[...] + s0b[...] + gs0[...]))
    h01 = h0[...] + h1
    h01_o[...] = h01
    g1 = _mm(h01, wg1[...]) + bg1[...]
    gs1_o[...] = _mm(d, g1)


def _tc3_body(s1a, s1b, gs1, h01, dinv, wc1, bc1, wc2, bc2, out_o):
    d = _diag(dinv[...][0])
    h2 = jax.nn.relu(_mm(d, s1a[...] + s1b[...] + gs1[...]))
    f = h01[...] + h2
    r = jax.nn.relu(_mm(f, wc1[...]) + bc1[...])
    out_o[...] = _mm(r, wc2[...]) + bc2[...]


_blk = pl.BlockSpec((F, F), lambda i: (i, 0))
_row = pl.BlockSpec((1, 1, F), lambda i: (i, 0, 0))
_w = pl.BlockSpec((F, F), lambda i: (0, 0))
_b = pl.BlockSpec((1, F), lambda i: (0, 0))

_nf32 = jax.ShapeDtypeStruct((N_PAD, F), jnp.float32)

_tc1 = pl.pallas_call(
    _tc1_body, grid=(NB,),
    in_specs=[_blk, _row, _row, _w, _b, _w, _b],
    out_specs=[_row, _blk, _blk],
    out_shape=[jax.ShapeDtypeStruct((NB, 1, F), jnp.float32), _nf32, _nf32],
)

_tc2 = pl.pallas_call(
    _tc2_body, grid=(NB,),
    in_specs=[_blk, _blk, _blk, _blk, _row, _w, _b],
    out_specs=[_blk, _blk],
    out_shape=[_nf32, _nf32],
)

_tc3 = pl.pallas_call(
    _tc3_body, grid=(79,),
    in_specs=[_blk, _blk, _blk, _blk, _row, _w, _b, _w, _b],
    out_specs=[_blk],
    out_shape=[jax.ShapeDtypeStruct((N, F), jnp.float32)],
)


def kernel(x, edge_index, W1, b1, Wg0, bg0, Wg1, bg1, Wc1, bc1, Wc2, bc2):
    src = edge_index[0]
    dst = edge_index[1]
    pad = E_PAD - E
    pad_idx = DUMMY + (jnp.arange(pad, dtype=jnp.int32) % (N_PAD - N))
    src_p = jnp.concatenate([src, pad_idx])
    dst_p = jnp.concatenate([dst, pad_idx])
    src_r = src_p.reshape(NW, CHUNKS, F)
    dst_r = dst_p.reshape(NW, CHUNKS, F)
    dst_r32 = dst_r
    x_p = jnp.zeros((N_PAD, F), jnp.float32).at[:N].set(x)

    deg = _deg_kernel(jnp.eye(F, dtype=jnp.float32), dst_r32)
    b1r = b1.reshape(1, F)
    bg0r = bg0.reshape(1, F)
    bg1r = bg1.reshape(1, F)
    bc1r = bc1.reshape(1, F)
    bc2r = bc2.reshape(1, F)
    dega = deg[0].reshape(NB, 1, F)
    degb = deg[1].reshape(NB, 1, F)
    dinv, h0, gs0 = _tc1(x_p, dega, degb, W1, b1r, Wg0, bg0r)
    s0 = _agg_kernel(gs0, src_r, dst_r)
    h01, gs1 = _tc2(s0[0], s0[1], gs0, h0, dinv, Wg1, bg1r)
    s1 = _agg_kernel(gs1, src_r, dst_r)
    out, = _tc3(s1[0], s1[1], gs1, h01, dinv, Wc1, bc1r, Wc2, bc2r)
    return out


# speedup vs baseline: 1.0005x; 1.0005x over previous
import functools

import jax
import jax.numpy as jnp
from jax import lax
from jax.experimental import pallas as pl
from jax.experimental.pallas import tpu as pltpu
from jax.experimental.pallas import tpu_sc as plsc

N = 10000
E = 320000
F = 128
N_PAD = 10240
NB = N_PAD // F
NW = 32
EPT = 10240
CHUNKS = EPT // F
NBUF = 2
DNBUF = 4
E_PAD = NW * EPT
DUMMY = N


@functools.cache
def _build_deg_kernel():
    return functools.partial(
        pl.kernel,
        out_type=jax.ShapeDtypeStruct((2, NB, F), jnp.float32),
        mesh=plsc.VectorSubcoreMesh(core_axis_name="c", subcore_axis_name="s"),
        scratch_types=[
        pltpu.VMEM((CHUNKS, F), jnp.int32),
        pltpu.VMEM((DNBUF, F), jnp.int32),
        pltpu.VMEM((DNBUF, F), jnp.int32),
        [pltpu.VMEM((F, F), jnp.float32)] * DNBUF,
        pltpu.VMEM((NB, F), jnp.float32),
        pltpu.VMEM_SHARED((F, F), jnp.float32),
        pltpu.VMEM_SHARED((NB, F), jnp.float32),
        [pltpu.SemaphoreType.DMA] * DNBUF,
        ],
    )(_deg_body)


def _deg_body(eye_hbm, dst_hbm, out_hbm,
                dst_v, cidx_v, ridx_v, rows, zb_v, eye_sh, deg_sh, sems):
    c = lax.axis_index("c")
    s = lax.axis_index("s")
    wid = c * 16 + s
    z16 = jnp.zeros((16,), jnp.float32)

    def zbody(i, carry):
        for k in range(8):
            zb_v[i, pl.ds(k * 16, 16)] = z16
        return carry

    lax.fori_loop(0, NB, zbody, 0)

    @pl.when(s == 0)
    def _():
        pltpu.sync_copy(zb_v, deg_sh)

    @pl.when(s == 1)
    def _():
        pltpu.sync_copy(eye_hbm, eye_sh)

    pltpu.sync_copy(dst_hbm.at[wid], dst_v)

    def setidx(chunk, b):
        for k in range(8):
            sl = pl.ds(k * 16, 16)
            d = dst_v[chunk, sl]
            cidx_v[b, sl] = lax.bitwise_and(d, F - 1)
            ridx_v[b, sl] = lax.shift_right_logical(d, 7)

    for b in range(DNBUF):
        setidx(b, b)
        pltpu.async_copy(eye_sh.at[cidx_v.at[b]], rows[b], sems[b])
    plsc.subcore_barrier()

    def body(g, carry):
        for b in range(DNBUF):
            chunk = g * DNBUF + b
            pltpu.make_async_copy(eye_sh.at[cidx_v.at[b]], rows[b],
                                  sems[b]).wait()
            pltpu.sync_copy(rows[b], deg_sh.at[ridx_v.at[b]], add=True)
            nxt = chunk + DNBUF

            @pl.when(nxt < CHUNKS)
            def _():
                setidx(nxt, b)
                pltpu.async_copy(eye_sh.at[cidx_v.at[b]], rows[b], sems[b])
        return carry

    lax.fori_loop(0, CHUNKS // DNBUF, body, 0)
    plsc.subcore_barrier()

    @pl.when(s == 0)
    def _():
        pltpu.sync_copy(deg_sh, out_hbm.at[c])


@functools.cache
def _build_agg_kernel():
    return functools.partial(
        pl.kernel,
        out_type=jax.ShapeDtypeStruct((2, N_PAD, F), jnp.float32),
        mesh=plsc.VectorSubcoreMesh(core_axis_name="c", subcore_axis_name="s"),
        scratch_types=[
        pltpu.VMEM((CHUNKS, F), jnp.int32),
        pltpu.VMEM((CHUNKS, F), jnp.int32),
        [pltpu.VMEM((F, F), jnp.float32)] * NBUF,
        pltpu.VMEM_SHARED((N_PAD, F), jnp.float32),
        [pltpu.SemaphoreType.DMA] * NBUF,
        ],
    )(_agg_body)


def _agg_body(table_hbm, src_hbm, dst_hbm, out_hbm,
                src_v, dst_v, rows, acc_sh, sems):
    c = lax.axis_index("c")
    s = lax.axis_index("s")
    wid = c * 16 + s
    z16 = jnp.zeros((16,), jnp.float32)

    def zbody(i, carry):
        for k in range(8):
            rows[0][i, pl.ds(k * 16, 16)] = z16
        return carry

    lax.fori_loop(0, F, zbody, 0)
    rows_per_tile = N_PAD // 16

    def zsh(k, carry):
        pltpu.sync_copy(rows[0],
                        acc_sh.at[pl.ds(s * rows_per_tile + k * F, F)])
        return carry

    lax.fori_loop(0, rows_per_tile // F, zsh, 0)
    pltpu.sync_copy(src_hbm.at[wid], src_v)
    pltpu.sync_copy(dst_hbm.at[wid], dst_v)
    plsc.subcore_barrier()

    def body(chunk, carry):
        pltpu.async_copy(table_hbm.at[src_v.at[chunk]], rows[0],
                         sems[0]).wait()
        pltpu.sync_copy(rows[0], acc_sh.at[dst_v.at[chunk]], add=True)
        return carry

    lax.fori_loop(0, CHUNKS, body, 0)
    plsc.subcore_barrier()
    pltpu.sync_copy(acc_sh.at[pl.ds(s * rows_per_tile, rows_per_tile)],
                    out_hbm.at[c, pl.ds(s * rows_per_tile, rows_per_tile)])


def _diag(dinv_row):
    row = lax.broadcasted_iota(jnp.int32, (F, F), 0)
    col = lax.broadcasted_iota(jnp.int32, (F, F), 1)
    return jnp.where(row == col, jnp.broadcast_to(dinv_row, (F, F)), 0.0)


def _mm(a, b):
    return jax.lax.dot_general(a, b, (((1,), (0,)), ((), ())),
                               preferred_element_type=jnp.float32)


def _tc1_body(x, dega, degb, w1, b1, wg0, bg0, dinv_o, h0_o, gs0_o):
    dinv = lax.rsqrt(dega[...] + degb[...] + 1.0)
    dinv_o[...] = dinv
    h0 = _mm(x[...], w1[...]) + b1[...]
    h0_o[...] = h0
    g0 = _mm(h0, wg0[...]) + bg0[...]
    gs0_o[...] = _mm(_diag(dinv[0]), g0)


def _tc2_body(s0a, s0b, gs0, h0, dinv, wg1, bg1, h01_o, gs1_o):
    d = _diag(dinv[...][0])
    h1 = jax.nn.relu(_mm(d, s0a[...] + s0b[...] + gs0[...]))
    h01 = h0[...] + h1
    h01_o[...] = h01
    g1 = _mm(h01, wg1[...]) + bg1[...]
    gs1_o[...] = _mm(d, g1)


def _tc3_body(s1a, s1b, gs1, h01, dinv, wc1, bc1, wc2, bc2, out_o):
    d = _diag(dinv[...][0])
    h2 = jax.nn.relu(_mm(d, s1a[...] + s1b[...] + gs1[...]))
    f = h01[...] + h2
    r = jax.nn.relu(_mm(f, wc1[...]) + bc1[...])
    out_o[...] = _mm(r, wc2[...]) + bc2[...]


_blk = pl.BlockSpec((F, F), lambda i: (i, 0))
_row = pl.BlockSpec((1, 1, F), lambda i: (i, 0, 0))
_w = pl.BlockSpec((F, F), lambda i: (0, 0))
_b = pl.BlockSpec((1, F), lambda i: (0, 0))

_nf32 = jax.ShapeDtypeStruct((N_PAD, F), jnp.float32)

_tc1 = pl.pallas_call(
    _tc1_body, grid=(NB,),
    in_specs=[_blk, _row, _row, _w, _b, _w, _b],
    out_specs=[_row, _blk, _blk],
    out_shape=[jax.ShapeDtypeStruct((NB, 1, F), jnp.float32), _nf32, _nf32],
)

_tc2 = pl.pallas_call(
    _tc2_body, grid=(NB,),
    in_specs=[_blk, _blk, _blk, _blk, _row, _w, _b],
    out_specs=[_blk, _blk],
    out_shape=[_nf32, _nf32],
)

_tc3 = pl.pallas_call(
    _tc3_body, grid=(79,),
    in_specs=[_blk, _blk, _blk, _blk, _row, _w, _b, _w, _b],
    out_specs=[_blk],
    out_shape=[jax.ShapeDtypeStruct((N, F), jnp.float32)],
)


def kernel(x, edge_index, W1, b1, Wg0, bg0, Wg1, bg1, Wc1, bc1, Wc2, bc2):
    src = edge_index[0]
    dst = edge_index[1]
    pad = E_PAD - E
    pad_idx = DUMMY + (jnp.arange(pad, dtype=jnp.int32) % (N_PAD - N))
    src_p = jnp.concatenate([src, pad_idx])
    dst_p = jnp.concatenate([dst, pad_idx])
    src_r = src_p.reshape(NW, CHUNKS, F)
    dst_r = dst_p.reshape(NW, CHUNKS, F)
    dst_r32 = dst_r
    x_p = jnp.zeros((N_PAD, F), jnp.float32).at[:N].set(x)

    deg = _build_deg_kernel()(jnp.eye(F, dtype=jnp.float32), dst_r32)
    b1r = b1.reshape(1, F)
    bg0r = bg0.reshape(1, F)
    bg1r = bg1.reshape(1, F)
    bc1r = bc1.reshape(1, F)
    bc2r = bc2.reshape(1, F)
    dega = deg[0].reshape(NB, 1, F)
    degb = deg[1].reshape(NB, 1, F)
    dinv, h0, gs0 = _tc1(x_p, dega, degb, W1, b1r, Wg0, bg0r)
    agg = _build_agg_kernel()
    s0 = agg(gs0, src_r, dst_r)
    h01, gs1 = _tc2(s0[0], s0[1], gs0, h0, dinv, Wg1, bg1r)
    s1 = agg(gs1, src_r, dst_r)
    out, = _tc3(s1[0], s1[1], gs1, h01, dinv, Wc1, bc1r, Wc2, bc2r)
    return out
